# baseline (speedup 1.0000x reference)
import functools

import numpy as np
import jax
import jax.numpy as jnp
from jax import lax
from jax.experimental import pallas as pl
from jax.experimental.pallas import tpu as pltpu
from jax.experimental.pallas import tpu_sc as plsc

_B, _D, _K = 512, 256, 1024
_SIDE = 32
_SCALE = float(1.0 / (2.0 * np.sqrt(2.0 * np.pi)))
_NC, _NS, _L = 2, 16, 16
_NW = _NC * _NS
_RPW = _B // _NW


def _tc_body(x_ref, w_ref, n2ec_ref, er_ref):
    x = x_ref[...]
    w = w_ref[...]
    xw = lax.dot_general(
        x, w, (((1,), (0,)), ((), ())),
        preferred_element_type=jnp.float32,
        precision=lax.Precision.HIGHEST,
    )
    x2 = jnp.sum(x * x, axis=1, keepdims=True)
    w2 = jnp.sum(w * w, axis=0, keepdims=True)
    norms2 = (x2 + w2) - 2.0 * xw
    kidx = lax.broadcasted_iota(jnp.int32, (_B, _K), 1)
    minv = jnp.min(norms2, axis=1, keepdims=True)
    wta = jnp.min(jnp.where(norms2 <= minv, kidx, _K), axis=1, keepdims=True)

    wr = (wta >> 5).astype(jnp.float32)
    wc = (wta & 31).astype(jnp.float32)
    dc = (kidx & 31).astype(jnp.float32) - wc
    n2ec_ref[...] = norms2 * (jnp.exp(-0.125 * (dc * dc)) * _SCALE)

    j32 = lax.broadcasted_iota(jnp.int32, (_B, _SIDE), 1).astype(jnp.float32)
    dr = j32 - wr
    er_ref[...] = jnp.exp(-0.125 * (dr * dr))


_BLK = 4
_NBLK = _RPW // _BLK


@functools.partial(
    pl.kernel,
    mesh=plsc.VectorSubcoreMesh(core_axis_name="c", subcore_axis_name="s"),
    out_type=jax.ShapeDtypeStruct((_B, _K), jnp.float32),
    scratch_types=[
        pltpu.VMEM((_RPW, _SIDE), jnp.float32),
        pltpu.VMEM((2, _BLK, _K), jnp.float32),
        pltpu.VMEM((_RPW, _K), jnp.float32),
        pltpu.SemaphoreType.DMA,
        pltpu.SemaphoreType.DMA,
        pltpu.SemaphoreType.DMA,
    ],
    compiler_params=pltpu.CompilerParams(
        needs_layout_passes=False,
        skip_device_barrier=True,
    ),
)
def _sc_radial(n2ec_hbm, er_hbm, out_hbm, er_v, n2_v, out_v,
               in_sem0, in_sem1, out_sem):
    wid = lax.axis_index("s") * _NC + lax.axis_index("c")
    base = wid * _RPW
    in_sems = (in_sem0, in_sem1)
    pltpu.sync_copy(er_hbm.at[pl.ds(base, _RPW)], er_v)

    h_in = [pltpu.async_copy(
        n2ec_hbm.at[pl.ds(base, _BLK)], n2_v.at[0], in_sems[0])]
    h_out = []
    for b in range(_NBLK):
        buf = b & 1
        if b + 1 < _NBLK:
            h_in.append(pltpu.async_copy(
                n2ec_hbm.at[pl.ds(base + (b + 1) * _BLK, _BLK)],
                n2_v.at[(b + 1) & 1], in_sems[(b + 1) & 1]))
        h_in[b].wait()
        for j in range(_BLK):
            row = b * _BLK + j
            er_lo = er_v[row, pl.ds(0, _L)]
            er_hi = er_v[row, pl.ds(_L, _L)]
            for r in range(_SIDE):
                half = er_lo if r < _L else er_hi
                er_b = jnp.broadcast_to(half[r % _L], (_L,))
                off = r * _SIDE
                out_v[row, pl.ds(off, _L)] = (
                    n2_v[buf, j, pl.ds(off, _L)] * er_b)
                out_v[row, pl.ds(off + _L, _L)] = (
                    n2_v[buf, j, pl.ds(off + _L, _L)] * er_b)
        h_out.append(pltpu.async_copy(
            out_v.at[pl.ds(b * _BLK, _BLK)],
            out_hbm.at[pl.ds(base + b * _BLK, _BLK)], out_sem))
    for h in h_out:
        h.wait()


def kernel(x, kernel):
    n2ec, er = pl.pallas_call(
        _tc_body,
        out_shape=(
            jax.ShapeDtypeStruct((_B, _K), jnp.float32),
            jax.ShapeDtypeStruct((_B, _SIDE), jnp.float32),
        ),
    )(x, kernel)
    return _sc_radial(n2ec, er)


# speedup vs baseline: 1.0120x; 1.0120x over previous
import functools

import numpy as np
import jax
import jax.numpy as jnp
from jax import lax
from jax.experimental import pallas as pl
from jax.experimental.pallas import tpu as pltpu
from jax.experimental.pallas import tpu_sc as plsc

_B, _D, _K = 512, 256, 1024
_SIDE = 32
_SCALE = float(1.0 / (2.0 * np.sqrt(2.0 * np.pi)))
_NC, _NS, _L = 2, 16, 16
_NW = _NC * _NS
_RPW = _B // _NW


def _tc_body(x_ref, w_ref, n2ec_ref, er_ref):
    x = x_ref[...]
    w = w_ref[...]
    xw = lax.dot_general(
        x, w, (((1,), (0,)), ((), ())),
        preferred_element_type=jnp.float32,
        precision=lax.Precision.HIGHEST,
    )
    x2 = jnp.sum(x * x, axis=1, keepdims=True)
    w2 = jnp.sum(w * w, axis=0, keepdims=True)
    norms2 = (x2 + w2) - 2.0 * xw
    kidx = lax.broadcasted_iota(jnp.int32, (_B, _K), 1)
    minv = jnp.min(norms2, axis=1, keepdims=True)
    wta = jnp.min(jnp.where(norms2 <= minv, kidx, _K), axis=1, keepdims=True)

    wr = (wta >> 5).astype(jnp.float32)
    wc = (wta & 31).astype(jnp.float32)
    dc = (kidx & 31).astype(jnp.float32) - wc
    n2ec_ref[...] = norms2 * (jnp.exp(-0.125 * (dc * dc)) * _SCALE)

    j32 = lax.broadcasted_iota(jnp.int32, (_B, _SIDE), 1).astype(jnp.float32)
    dr = j32 - wr
    er_ref[...] = jnp.exp(-0.125 * (dr * dr))


@functools.partial(
    pl.kernel,
    mesh=plsc.VectorSubcoreMesh(core_axis_name="c", subcore_axis_name="s"),
    out_type=jax.ShapeDtypeStruct((_B, _K), jnp.float32),
    scratch_types=[
        pltpu.VMEM((_RPW, _SIDE), jnp.float32),
        pltpu.VMEM((_RPW, _K), jnp.float32),
        pltpu.VMEM((_RPW, _K), jnp.float32),
    ],
    compiler_params=pltpu.CompilerParams(
        needs_layout_passes=False,
        skip_device_barrier=True,
    ),
)
def _sc_radial(n2ec_hbm, er_hbm, out_hbm, er_v, n2_v, out_v):
    wid = lax.axis_index("s") * _NC + lax.axis_index("c")
    base = wid * _RPW
    pltpu.sync_copy(er_hbm.at[pl.ds(base, _RPW)], er_v)
    pltpu.sync_copy(n2ec_hbm.at[pl.ds(base, _RPW)], n2_v)

    def row_body(i, carry):
        er_lo = er_v[i, pl.ds(0, _L)]
        er_hi = er_v[i, pl.ds(_L, _L)]
        for r in range(_SIDE):
            half = er_lo if r < _L else er_hi
            er_b = jnp.broadcast_to(half[r % _L], (_L,))
            off = r * _SIDE
            out_v[i, pl.ds(off, _L)] = n2_v[i, pl.ds(off, _L)] * er_b
            out_v[i, pl.ds(off + _L, _L)] = n2_v[i, pl.ds(off + _L, _L)] * er_b
        return carry

    lax.fori_loop(0, _RPW, row_body, 0, unroll=2)
    pltpu.sync_copy(out_v, out_hbm.at[pl.ds(base, _RPW)])


def kernel(x, kernel):
    n2ec, er = pl.pallas_call(
        _tc_body,
        out_shape=(
            jax.ShapeDtypeStruct((_B, _K), jnp.float32),
            jax.ShapeDtypeStruct((_B, _SIDE), jnp.float32),
        ),
    )(x, kernel)
    return _sc_radial(n2ec, er)


# speedup vs baseline: 1.0374x; 1.0251x over previous
import functools

import numpy as np
import jax
import jax.numpy as jnp
from jax import lax
from jax.experimental import pallas as pl
from jax.experimental.pallas import tpu as pltpu
from jax.experimental.pallas import tpu_sc as plsc

_B, _D, _K = 512, 256, 1024
_SIDE = 32
_SCALE = float(1.0 / (2.0 * np.sqrt(2.0 * np.pi)))
_NC, _NS, _L = 2, 16, 16
_NW = _NC * _NS
_RPW = _B // _NW


def _tc_body(x_ref, w_ref, n2ec_ref, er_ref):
    x = x_ref[...]
    w = w_ref[...]
    xw = lax.dot_general(
        x, w, (((1,), (0,)), ((), ())),
        preferred_element_type=jnp.float32,
        precision=lax.Precision.HIGHEST,
    )
    x2 = jnp.sum(x * x, axis=1, keepdims=True)
    w2 = jnp.sum(w * w, axis=0, keepdims=True)
    norms2 = (x2 + w2) - 2.0 * xw
    kidx = lax.broadcasted_iota(jnp.int32, (_B, _K), 1)
    minv = jnp.min(norms2, axis=1, keepdims=True)
    wta = jnp.min(jnp.where(norms2 <= minv, kidx, _K), axis=1, keepdims=True)

    wr = (wta >> 5).astype(jnp.float32)
    wc = (wta & 31).astype(jnp.float32)
    dc = (kidx & 31).astype(jnp.float32) - wc
    n2ec_ref[...] = norms2 * (jnp.exp(-0.125 * (dc * dc)) * _SCALE)

    j32 = lax.broadcasted_iota(jnp.int32, (_B, _SIDE), 1).astype(jnp.float32)
    dr = j32 - wr
    er_ref[...] = jnp.exp(-0.125 * (dr * dr))


@functools.partial(
    pl.kernel,
    mesh=plsc.VectorSubcoreMesh(core_axis_name="c", subcore_axis_name="s"),
    out_type=jax.ShapeDtypeStruct((_B, _K), jnp.float32),
    scratch_types=[
        pltpu.VMEM((_RPW, _SIDE), jnp.float32),
        pltpu.VMEM((_RPW, _K), jnp.float32),
        pltpu.VMEM((_RPW, _K), jnp.float32),
    ],
    compiler_params=pltpu.CompilerParams(
        needs_layout_passes=False,
    ),
)
def _sc_radial(n2ec_hbm, er_hbm, out_hbm, er_v, n2_v, out_v):
    wid = lax.axis_index("s") * _NC + lax.axis_index("c")
    base = wid * _RPW
    pltpu.sync_copy(er_hbm.at[pl.ds(base, _RPW)], er_v)
    pltpu.sync_copy(n2ec_hbm.at[pl.ds(base, _RPW)], n2_v)

    def row_body(i, carry):
        er_lo = er_v[i, pl.ds(0, _L)]
        er_hi = er_v[i, pl.ds(_L, _L)]
        for r in range(_SIDE):
            half = er_lo if r < _L else er_hi
            er_b = jnp.broadcast_to(half[r % _L], (_L,))
            off = r * _SIDE
            out_v[i, pl.ds(off, _L)] = n2_v[i, pl.ds(off, _L)] * er_b
            out_v[i, pl.ds(off + _L, _L)] = n2_v[i, pl.ds(off + _L, _L)] * er_b
        return carry

    lax.fori_loop(0, _RPW, row_body, 0)
    pltpu.sync_copy(out_v, out_hbm.at[pl.ds(base, _RPW)])


def kernel(x, kernel):
    n2ec, er = pl.pallas_call(
        _tc_body,
        out_shape=(
            jax.ShapeDtypeStruct((_B, _K), jnp.float32),
            jax.ShapeDtypeStruct((_B, _SIDE), jnp.float32),
        ),
    )(x, kernel)
    return _sc_radial(n2ec, er)


# speedup vs baseline: 1.0804x; 1.0415x over previous
import functools

import numpy as np
import jax
import jax.numpy as jnp
from jax import lax
from jax.experimental import pallas as pl
from jax.experimental.pallas import tpu as pltpu
from jax.experimental.pallas import tpu_sc as plsc

_B, _D, _K = 512, 256, 1024
_SIDE = 32
_SCALE = float(1.0 / (2.0 * np.sqrt(2.0 * np.pi)))
_NC, _NS, _L = 2, 16, 16
_NW = _NC * _NS
_RPW = _B // _NW


def _tc_body(x_ref, w_ref, n2ec_ref, er_ref):
    x = x_ref[...]
    w = w_ref[...]
    xw = lax.dot_general(
        x, w, (((1,), (0,)), ((), ())),
        preferred_element_type=jnp.float32,
        precision=lax.Precision.HIGHEST,
    )
    x2 = jnp.sum(x * x, axis=1, keepdims=True)
    w2 = jnp.sum(w * w, axis=0, keepdims=True)
    norms2 = (x2 + w2) - 2.0 * xw
    kidx = lax.broadcasted_iota(jnp.int32, (_B, _K), 1)
    minv = jnp.min(norms2, axis=1, keepdims=True)
    wta = jnp.min(jnp.where(norms2 <= minv, kidx, _K), axis=1, keepdims=True)

    wr = (wta >> 5).astype(jnp.float32)
    wc = (wta & 31).astype(jnp.float32)
    dc = (kidx & 31).astype(jnp.float32) - wc
    n2ec_ref[...] = norms2 * (jnp.exp(-0.125 * (dc * dc)) * _SCALE)

    j32 = lax.broadcasted_iota(jnp.int32, (_B, _SIDE), 1).astype(jnp.float32)
    dr = j32 - wr
    er_ref[...] = jnp.exp(-0.125 * (dr * dr))


@functools.partial(
    pl.kernel,
    mesh=plsc.VectorSubcoreMesh(core_axis_name="c", subcore_axis_name="s"),
    out_type=jax.ShapeDtypeStruct((_B, _K), jnp.float32),
    scratch_types=[
        pltpu.VMEM((_RPW, _SIDE), jnp.float32),
        pltpu.VMEM((_RPW, _K), jnp.float32),
        pltpu.VMEM((_RPW, _K), jnp.float32),
        pltpu.SemaphoreType.DMA,
        pltpu.SemaphoreType.DMA,
        pltpu.SemaphoreType.DMA,
    ],
    compiler_params=pltpu.CompilerParams(
        needs_layout_passes=False,
    ),
)
def _sc_radial(n2ec_hbm, er_hbm, out_hbm, er_v, n2_v, out_v,
               sem_a, sem_b, sem_out):
    wid = lax.axis_index("s") * _NC + lax.axis_index("c")
    base = wid * _RPW
    half_rows = _RPW // 2
    h_a = pltpu.async_copy(
        n2ec_hbm.at[pl.ds(base, half_rows)],
        n2_v.at[pl.ds(0, half_rows)], sem_a)
    h_b = pltpu.async_copy(
        n2ec_hbm.at[pl.ds(base + half_rows, half_rows)],
        n2_v.at[pl.ds(half_rows, half_rows)], sem_b)
    pltpu.sync_copy(er_hbm.at[pl.ds(base, _RPW)], er_v)

    def row_body(i, carry):
        er_lo = er_v[i, pl.ds(0, _L)]
        er_hi = er_v[i, pl.ds(_L, _L)]
        for r in range(_SIDE):
            half = er_lo if r < _L else er_hi
            er_b = jnp.broadcast_to(half[r % _L], (_L,))
            off = r * _SIDE
            out_v[i, pl.ds(off, _L)] = n2_v[i, pl.ds(off, _L)] * er_b
            out_v[i, pl.ds(off + _L, _L)] = n2_v[i, pl.ds(off + _L, _L)] * er_b
        return carry

    h_a.wait()
    lax.fori_loop(0, half_rows, row_body, 0)
    h_out = pltpu.async_copy(
        out_v.at[pl.ds(0, half_rows)],
        out_hbm.at[pl.ds(base, half_rows)], sem_out)
    h_b.wait()
    lax.fori_loop(half_rows, _RPW, row_body, 0)
    h_out.wait()
    pltpu.sync_copy(
        out_v.at[pl.ds(half_rows, half_rows)],
        out_hbm.at[pl.ds(base + half_rows, half_rows)])


def kernel(x, kernel):
    n2ec, er = pl.pallas_call(
        _tc_body,
        out_shape=(
            jax.ShapeDtypeStruct((_B, _K), jnp.float32),
            jax.ShapeDtypeStruct((_B, _SIDE), jnp.float32),
        ),
    )(x, kernel)
    return _sc_radial(n2ec, er)
